# baseline (speedup 1.0000x reference)
import functools

import jax
import jax.numpy as jnp
from jax import lax
from jax.experimental import pallas as pl
from jax.experimental.pallas import tpu as pltpu
from jax.experimental.pallas import tpu_sc as plsc

NUM_FIELDS = 26
ROWS_PER_FIELD = 100000
EMBED = 16
BATCH = 16384
N = BATCH * NUM_FIELDS

LANES = 16
NUM_CORES = 2
NUM_SUBCORES = 16
NW = NUM_CORES * NUM_SUBCORES
PER_W = N // NW
CHUNK = 128
NCHUNK = PER_W // CHUNK
VECS = CHUNK // LANES
NBUF = 4
NGROUP = NCHUNK // NBUF


def _sc_lookup(x_flat, table):
    mesh = plsc.VectorSubcoreMesh(core_axis_name="c", subcore_axis_name="s")

    @functools.partial(
        pl.kernel,
        mesh=mesh,
        out_type=jax.ShapeDtypeStruct((N, EMBED), jnp.float32),
        compiler_params=pltpu.CompilerParams(use_tc_tiling_on_sc=False),
        scratch_types=[
            pltpu.VMEM((NCHUNK, CHUNK), jnp.int32),
            pltpu.VMEM((NBUF, CHUNK, EMBED), jnp.float32),
            pltpu.SemaphoreType.DMA,
            pltpu.SemaphoreType.DMA,
            pltpu.SemaphoreType.DMA,
            pltpu.SemaphoreType.DMA,
        ],
    )
    def k(x_hbm, table_hbm, out_hbm, idx_v, rows_v, s0, s1, s2, s3):
        wid = lax.axis_index("s") * NUM_CORES + lax.axis_index("c")
        base = wid * PER_W
        sems = (s0, s1, s2, s3)
        lane = lax.iota(jnp.int32, LANES)

        pltpu.sync_copy(x_hbm.at[wid], idx_v)

        def adjust(ci):
            row = idx_v.at[ci]
            for i in range(VECS):
                off = i * LANES
                f = lax.rem(ci * CHUNK + off + lane, NUM_FIELDS)
                row[pl.ds(off, LANES)] = (
                    row[pl.ds(off, LANES)] + f * ROWS_PER_FIELD
                )

        def start(ci, b):
            pltpu.async_copy(table_hbm.at[idx_v.at[ci]], rows_v.at[b], sems[b])

        def wait_and_out(ci, b):
            pltpu.make_async_copy(
                table_hbm.at[pl.ds(0, CHUNK)], rows_v.at[b], sems[b]
            ).wait()
            pltpu.sync_copy(
                rows_v.at[b], out_hbm.at[pl.ds(base + ci * CHUNK, CHUNK)]
            )

        for b in range(NBUF):
            adjust(b)
            start(b, b)

        def group(g, _):
            for b in range(NBUF):
                ci = g * NBUF + b
                wait_and_out(ci - NBUF, b)
                adjust(ci)
                start(ci, b)
            return 0

        lax.fori_loop(1, NGROUP, group, 0)

        for b in range(NBUF):
            wait_and_out((NGROUP - 1) * NBUF + b, b)

    return k(x_flat, table)


def kernel(x, table):
    x_flat = x.reshape(NW, NCHUNK, CHUNK)
    out = _sc_lookup(x_flat, table)
    return out.reshape(BATCH, NUM_FIELDS, EMBED)


# speedup vs baseline: 1.0060x; 1.0060x over previous
import functools

import jax
import jax.numpy as jnp
from jax import lax
from jax.experimental import pallas as pl
from jax.experimental.pallas import tpu as pltpu
from jax.experimental.pallas import tpu_sc as plsc

NUM_FIELDS = 26
ROWS_PER_FIELD = 100000
EMBED = 16
BATCH = 16384
N = BATCH * NUM_FIELDS

LANES = 16
NUM_CORES = 2
NUM_SUBCORES = 16
NW = NUM_CORES * NUM_SUBCORES
PER_W = N // NW
CHUNK = 1664
NCHUNK = PER_W // CHUNK
VECS = CHUNK // LANES
NBUF = 4
NGROUP = NCHUNK // NBUF


def _sc_lookup(x_flat, table):
    mesh = plsc.VectorSubcoreMesh(core_axis_name="c", subcore_axis_name="s")

    @functools.partial(
        pl.kernel,
        mesh=mesh,
        out_type=jax.ShapeDtypeStruct((N, EMBED), jnp.float32),
        compiler_params=pltpu.CompilerParams(use_tc_tiling_on_sc=False),
        scratch_types=[
            pltpu.VMEM((NCHUNK, CHUNK), jnp.int32),
            pltpu.VMEM((NBUF, CHUNK, EMBED), jnp.float32),
            pltpu.SemaphoreType.DMA((NBUF,)),
            pltpu.SemaphoreType.DMA((NBUF,)),
        ],
    )
    def k(x_hbm, table_hbm, out_hbm, idx_v, rows_v, gsem, osem):
        wid = lax.axis_index("s") * NUM_CORES + lax.axis_index("c")
        base = wid * PER_W
        lane = lax.iota(jnp.int32, LANES)

        pltpu.sync_copy(x_hbm.at[wid], idx_v)

        def adjust(ci):
            row = idx_v.at[ci]

            def body(i, _):
                off = i * LANES
                f = lax.rem(ci * CHUNK + off + lane, NUM_FIELDS)
                row[pl.ds(off, LANES)] = (
                    row[pl.ds(off, LANES)] + f * ROWS_PER_FIELD
                )
                return 0

            lax.fori_loop(0, VECS, body, 0)

        def start_gather(ci, b):
            pltpu.async_copy(
                table_hbm.at[idx_v.at[ci]], rows_v.at[b], gsem.at[b]
            )

        def wait_gather(b):
            pltpu.make_async_copy(
                table_hbm.at[pl.ds(0, CHUNK)], rows_v.at[b], gsem.at[b]
            ).wait()

        def start_out(ci, b):
            pltpu.async_copy(
                rows_v.at[b],
                out_hbm.at[pl.ds(base + ci * CHUNK, CHUNK)],
                osem.at[b],
            )

        def wait_out(ci, b):
            pltpu.make_async_copy(
                rows_v.at[b],
                out_hbm.at[pl.ds(base + ci * CHUNK, CHUNK)],
                osem.at[b],
            ).wait()

        def group(g, _):
            for b in range(NBUF):
                ci = g * NBUF + b

                @pl.when(g > 1)
                def _():
                    wait_out(ci - 2 * NBUF, b)

                adjust(ci)
                start_gather(ci, b)
            for b in range(NBUF):
                ci = g * NBUF + b
                wait_gather(b)
                start_out(ci, b)
            return 0

        lax.fori_loop(0, NGROUP, group, 0)

        for b in range(NBUF):
            wait_out((NGROUP - 2) * NBUF + b, b)
            wait_out((NGROUP - 1) * NBUF + b, b)

    return k(x_flat, table)


def kernel(x, table):
    x_flat = x.reshape(NW, NCHUNK, CHUNK)
    out = _sc_lookup(x_flat, table)
    return out.reshape(BATCH, NUM_FIELDS, EMBED)
